# baseline (speedup 1.0000x reference)
import functools

import jax
import jax.numpy as jnp
from jax import lax
from jax.experimental import pallas as pl
from jax.experimental.pallas import tpu as pltpu
from jax.experimental.pallas import tpu_sc as plsc

EMBED = 32
B = 16384
H1 = 128
H2 = 64
NC = 2
NS = 16
NW = NC * NS
BPW = B // NW
CH = 128
NCH = BPW // CH
SR = 128 // EMBED


def _sc_gather(uid, iid, Ug4, Ig4, Um4, Im4, ub, ib):
    mesh = plsc.VectorSubcoreMesh(core_axis_name="c", subcore_axis_name="s")

    @functools.partial(
        pl.kernel,
        mesh=mesh,
        compiler_params=pltpu.CompilerParams(use_tc_tiling_on_sc=False),
        out_type=[
            jax.ShapeDtypeStruct((B * EMBED,), jnp.float32),
            jax.ShapeDtypeStruct((B * EMBED,), jnp.float32),
            jax.ShapeDtypeStruct((B * EMBED,), jnp.float32),
            jax.ShapeDtypeStruct((B * EMBED,), jnp.float32),
            jax.ShapeDtypeStruct((B,), jnp.float32),
            jax.ShapeDtypeStruct((B,), jnp.float32),
        ],
        scratch_types=[
            pltpu.VMEM((BPW,), jnp.int32),
            pltpu.VMEM((BPW,), jnp.int32),
            pltpu.VMEM((BPW,), jnp.int32),
            pltpu.VMEM((BPW,), jnp.int32),
            pltpu.VMEM((BPW + 16,), jnp.int32),
            pltpu.VMEM((BPW + 16,), jnp.int32),
            pltpu.VMEM((CH, 128), jnp.float32),
            pltpu.VMEM((CH, 128), jnp.float32),
            pltpu.VMEM((CH, 128), jnp.float32),
            pltpu.VMEM((CH, 128), jnp.float32),
            pltpu.VMEM((2, CH * EMBED), jnp.float32),
            pltpu.VMEM((2, CH * EMBED), jnp.float32),
            pltpu.VMEM((2, CH * EMBED), jnp.float32),
            pltpu.VMEM((2, CH * EMBED), jnp.float32),
            pltpu.VMEM((BPW,), jnp.float32),
            pltpu.VMEM((BPW,), jnp.float32),
            pltpu.SemaphoreType.DMA,
            pltpu.SemaphoreType.DMA,
        ],
    )
    def k(uid_h, iid_h, ug_h, ig_h, um_h, im_h, ub_h, ib_h,
          oug, oig, oum, oim, obu, obi,
          uidx, iidx, usid, isid, uoff, ioff,
          vug, vig, vum, vim, cug, cig, cum, cim, vbu, vbi, gsem, wsem):
        wid = lax.axis_index("s") * NC + lax.axis_index("c")
        base = wid * BPW
        pltpu.sync_copy(uid_h.at[pl.ds(base, BPW)], uidx)
        pltpu.sync_copy(iid_h.at[pl.ds(base, BPW)], iidx)
        for t in range(BPW // 16):
            sl = pl.ds(t * 16, 16)
            u = uidx[sl]
            i = iidx[sl]
            usid[sl] = lax.shift_right_logical(u, 2)
            isid[sl] = lax.shift_right_logical(i, 2)
            uoff[sl] = lax.shift_left(jnp.bitwise_and(u, SR - 1), 5)
            ioff[sl] = lax.shift_left(jnp.bitwise_and(i, SR - 1), 5)
        bias = []
        for j in range(NCH):
            sl = pl.ds(j * CH, CH)
            bias.append(pltpu.async_copy(ub_h.at[uidx.at[sl]], vbu.at[sl], wsem))
            bias.append(pltpu.async_copy(ib_h.at[iidx.at[sl]], vbi.at[sl], wsem))
        pend = {0: [], 1: []}
        for j in range(NCH):
            sl = pl.ds(j * CH, CH)
            gath = [
                pltpu.async_copy(ug_h.at[usid.at[sl]], vug, gsem),
                pltpu.async_copy(um_h.at[usid.at[sl]], vum, gsem),
                pltpu.async_copy(ig_h.at[isid.at[sl]], vig, gsem),
                pltpu.async_copy(im_h.at[isid.at[sl]], vim, gsem),
            ]
            for c in gath:
                c.wait()
            par = j % 2
            for w in pend[par]:
                w.wait()
            pend[par] = []

            def body(r, carry, j=j, par=par):
                row = j * CH + r
                uo = uoff[pl.ds(row, 16)][0]
                io = ioff[pl.ds(row, 16)][0]
                for h in range(2):
                    dst = pl.ds(r * EMBED + h * 16, 16)
                    cug[par, dst] = vug[r, pl.ds(uo + h * 16, 16)]
                    cum[par, dst] = vum[r, pl.ds(uo + h * 16, 16)]
                    cig[par, dst] = vig[r, pl.ds(io + h * 16, 16)]
                    cim[par, dst] = vim[r, pl.ds(io + h * 16, 16)]
                return carry

            lax.fori_loop(0, CH, body, 0)
            ob = (base + j * CH) * EMBED
            osl = pl.ds(ob, CH * EMBED)
            pend[par] = [
                pltpu.async_copy(cug.at[par], oug.at[osl], wsem),
                pltpu.async_copy(cig.at[par], oig.at[osl], wsem),
                pltpu.async_copy(cum.at[par], oum.at[osl], wsem),
                pltpu.async_copy(cim.at[par], oim.at[osl], wsem),
            ]
        for b in bias:
            b.wait()
        bw = [
            pltpu.async_copy(vbu, obu.at[pl.ds(base, BPW)], wsem),
            pltpu.async_copy(vbi, obi.at[pl.ds(base, BPW)], wsem),
        ]
        for par in (0, 1):
            for w in pend[par]:
                w.wait()
        for w in bw:
            w.wait()

    return k(uid, iid, Ug4, Ig4, Um4, Im4, ub, ib)


def _tc_mlp(ug, ig, um, im, bu2, bi2, w1u, w1i, b1r, W2, b2r, wog, woh, bo):
    BLK = BPW
    G = B // BLK

    def body(ug_r, ig_r, um_r, im_r, bu_r, bi_r, w1u_r, w1i_r, b1_r,
             w2_r, b2_r, wog_r, woh_r, bo_r, out_r):
        g = ug_r[...] * ig_r[...]
        x1 = jnp.dot(um_r[...], w1u_r[...], preferred_element_type=jnp.float32)
        x1 = x1 + jnp.dot(im_r[...], w1i_r[...], preferred_element_type=jnp.float32)
        h1 = jnp.maximum(x1 + b1_r[...], 0.0)
        x2 = jnp.dot(h1, w2_r[...], preferred_element_type=jnp.float32)
        h2 = jnp.maximum(x2 + b2_r[...], 0.0)
        p = jnp.sum(g * wog_r[...], axis=1) + jnp.sum(h2 * woh_r[...], axis=1)
        out_r[...] = (p + bo_r[0]).reshape(1, 1, BLK) + bu_r[...] + bi_r[...]

    out = pl.pallas_call(
        body,
        grid=(G,),
        in_specs=[
            pl.BlockSpec((BLK, EMBED), lambda i: (i, 0)),
            pl.BlockSpec((BLK, EMBED), lambda i: (i, 0)),
            pl.BlockSpec((BLK, EMBED), lambda i: (i, 0)),
            pl.BlockSpec((BLK, EMBED), lambda i: (i, 0)),
            pl.BlockSpec((1, 1, BLK), lambda i: (i, 0, 0)),
            pl.BlockSpec((1, 1, BLK), lambda i: (i, 0, 0)),
            pl.BlockSpec((EMBED, H1), lambda i: (0, 0)),
            pl.BlockSpec((EMBED, H1), lambda i: (0, 0)),
            pl.BlockSpec((1, H1), lambda i: (0, 0)),
            pl.BlockSpec((H1, H2), lambda i: (0, 0)),
            pl.BlockSpec((1, H2), lambda i: (0, 0)),
            pl.BlockSpec((1, EMBED), lambda i: (0, 0)),
            pl.BlockSpec((1, H2), lambda i: (0, 0)),
            pl.BlockSpec(memory_space=pltpu.SMEM),
        ],
        out_specs=pl.BlockSpec((1, 1, BLK), lambda i: (i, 0, 0)),
        out_shape=jax.ShapeDtypeStruct((G, 1, BLK), jnp.float32),
    )(ug, ig, um, im, bu2, bi2, w1u, w1i, b1r, W2, b2r, wog, woh, bo)
    return out.reshape(B)


def kernel(user_ids, item_ids, Ug, Ig, Um, Im, Ub, Ib, W1, b1, W2, b2, Wo, bo):
    uid = user_ids.astype(jnp.int32)
    iid = item_ids.astype(jnp.int32)
    ugf, igf, umf, imf, bu, bi = _sc_gather(
        uid, iid,
        Ug.reshape(-1, 128), Ig.reshape(-1, 128),
        Um.reshape(-1, 128), Im.reshape(-1, 128),
        Ub.reshape(-1), Ib.reshape(-1))
    return _tc_mlp(
        ugf.reshape(B, EMBED), igf.reshape(B, EMBED),
        umf.reshape(B, EMBED), imf.reshape(B, EMBED),
        bu.reshape(B // BPW, 1, BPW), bi.reshape(B // BPW, 1, BPW),
        W1[:EMBED], W1[EMBED:], b1.reshape(1, H1),
        W2, b2.reshape(1, H2),
        Wo[:EMBED].reshape(1, EMBED), Wo[EMBED:].reshape(1, H2), bo)
